# baseline (speedup 1.0000x reference)
import functools
import math

import jax
import jax.numpy as jnp
from jax import lax
from jax.experimental import pallas as pl
from jax.experimental.pallas import tpu as pltpu, tpu_sc as plsc

N = 10000
E_EDGES = 320000
D = 128
D_FF = 256
N_EXP = 8
ROWW = D + 16
INV_SQRT_D = 1.0 / math.sqrt(float(D))

NC, NS, L = 2, 16, 16
NW = NC * NS
EDGES_PER_W = E_EDGES // NW
CHUNK = 80
N_CHUNKS = EDGES_PER_W // CHUNK
NPAD = 10240
ROWS_PER_TILE = NPAD // NS
ZROWS = 128


DROWS = NPAD // D
CH = 32
NCH = 316
NFULL = 312
LAST_OFF = EDGES_PER_W - CH


def _edge_kernel_body(h_hbm, src_hbm, dst_hbm, num_hbm, den_hbm,
                      num_sp, den_sp,
                      srcidx, dstidx, sdstb, hs0, hs1, hd0, hd1,
                      out0, out1, den_l, idx80, wbuf,
                      sa0, sa1, sb0, sb1, sc0, sc1,
                      sis0, sis1, sis2, sis3, sid0, sid1, sid2, sid3):
    c = lax.axis_index("c")
    s = lax.axis_index("s")
    wid = c * NS + s
    lane = lax.iota(jnp.int32, L)
    hs = [hs0, hs1]
    hd = [hd0, hd1]
    out = [out0, out1]
    sa = [sa0, sa1]
    sb = [sb0, sb1]
    scs = [sc0, sc1]
    sis = [sis0, sis1, sis2, sis3]
    sid = [sid0, sid1, sid2, sid3]

    def zden(r, _):
        for k in range(D // L):
            den_l[r, pl.ds(k * L, L)] = jnp.zeros((L,), jnp.float32)
        return 0
    lax.fori_loop(0, DROWS, zden, 0)

    def zout(r, _):
        for k in range(D // L):
            out0[r, pl.ds(k * L, L)] = jnp.zeros((L,), jnp.float32)
        return 0
    lax.fori_loop(0, CH, zout, 0)

    def fidx(i, _):
        idx80[pl.ds(i * L, L)] = i * L + lane
        return 0
    lax.fori_loop(0, DROWS // L, fidx, 0)

    base = wid * EDGES_PER_W

    def chunk_off(ci):
        return pl.multiple_of(base + jnp.minimum(ci * CH, LAST_OFF), 8)

    for q in range(4):
        pltpu.async_copy(src_hbm.at[pl.ds(chunk_off(q), CH)],
                         srcidx.at[q], sis[q])
        pltpu.async_copy(dst_hbm.at[pl.ds(chunk_off(q), CH)],
                         dstidx.at[q], sid[q])
    for b in range(2):
        pltpu.make_async_copy(src_hbm.at[pl.ds(chunk_off(b), CH)],
                              srcidx.at[b], sis[b]).wait()
        pltpu.make_async_copy(dst_hbm.at[pl.ds(chunk_off(b), CH)],
                              dstidx.at[b], sid[b]).wait()
        pltpu.async_copy(h_hbm.at[srcidx.at[b]], hs[b], sa[b])
        pltpu.async_copy(h_hbm.at[dstidx.at[b]], hd[b], sb[b])

    stripe0 = s * ROWS_PER_TILE
    for j in range(ROWS_PER_TILE // CH):
        r0 = pl.multiple_of(stripe0 + j * CH, 8)
        pltpu.sync_copy(out0, num_sp.at[pl.ds(r0, CH)])

    @pl.when(s == 0)
    def _():
        pltpu.sync_copy(out0, den_sp.at[pl.ds(0, CH)])
        pltpu.sync_copy(out0, den_sp.at[pl.ds(CH, CH)])
        pltpu.sync_copy(out0.at[pl.ds(0, 16)], den_sp.at[pl.ds(2 * CH, 16)])

    plsc.subcore_barrier()

    def outer_body(t, _):
        for q in range(4):
            ci = 4 * t + q
            b = q % 2
            pltpu.make_async_copy(h_hbm.at[srcidx.at[q]], hs[b], sa[b]).wait()
            pltpu.make_async_copy(h_hbm.at[dstidx.at[q]], hd[b], sb[b]).wait()

            @pl.when(ci >= 2)
            def _():
                pltpu.make_async_copy(
                    out[b], num_sp.at[sdstb.at[b]], scs[b]).wait()

            for g in range(CH // L):
                sdstb[b, pl.ds(g * L, L)] = dstidx[q, pl.ds(g * L, L)]

            def wpass_body(g, _, b=b, ci=ci):
                e0 = g * L
                dv = sdstb[b, pl.ds(e0, L)]
                vcond = jnp.logical_or(
                    ci < NFULL,
                    jnp.logical_and(ci == NFULL, g == CH // L - 1))
                validf = jnp.where(vcond, 1.0, 0.0)
                for u in range(L):
                    e = e0 + u
                    acc = (hs[b][e, pl.ds(0, L)] * hd[b][e, pl.ds(0, L)])
                    for k in range(1, D // L):
                        acc = acc + (hs[b][e, pl.ds(k * L, L)]
                                     * hd[b][e, pl.ds(k * L, L)])
                    sco = jnp.sum(acc) * INV_SQRT_D
                    wv = jnp.exp(jnp.full((L,), sco, jnp.float32)) * validf
                    wbuf[e] = wv
                    d_e = dv[u]
                    row = lax.shift_right_logical(d_e, 7)
                    seg = lax.shift_left(lax.shift_right_logical(
                        jnp.bitwise_and(d_e, 127), 4), 4)
                    sub = jnp.bitwise_and(d_e, 15)
                    cur = den_l[row, pl.ds(seg, L)]
                    den_l[row, pl.ds(seg, L)] = (
                        cur + jnp.where(lane == sub, wv, 0.0))
                return 0

            lax.fori_loop(0, CH // L, wpass_body, 0)

            def spass_body(g, _, b=b):
                e0 = g * L
                for u in range(L):
                    e = e0 + u
                    wv = wbuf[e]
                    for k in range(D // L):
                        out[b][e, pl.ds(k * L, L)] = (
                            hs[b][e, pl.ds(k * L, L)] * wv)
                return 0

            lax.fori_loop(0, CH // L, spass_body, 0)

            pltpu.async_copy(out[b], num_sp.at[sdstb.at[b]], scs[b],
                             add=True)

            @pl.when(ci + 4 < NCH)
            def _():
                off4 = chunk_off(ci + 4)
                pltpu.async_copy(src_hbm.at[pl.ds(off4, CH)],
                                 srcidx.at[q], sis[q])
                pltpu.async_copy(dst_hbm.at[pl.ds(off4, CH)],
                                 dstidx.at[q], sid[q])

            @pl.when(ci + 2 < NCH)
            def _():
                q2 = (q + 2) % 4
                off2 = chunk_off(ci + 2)
                pltpu.make_async_copy(src_hbm.at[pl.ds(off2, CH)],
                                      srcidx.at[q2], sis[q2]).wait()
                pltpu.make_async_copy(dst_hbm.at[pl.ds(off2, CH)],
                                      dstidx.at[q2], sid[q2]).wait()
                pltpu.async_copy(h_hbm.at[srcidx.at[q2]], hs[b], sa[b])
                pltpu.async_copy(h_hbm.at[dstidx.at[q2]], hd[b], sb[b])
        return 0

    lax.fori_loop(0, NCH // 4, outer_body, 0)
    for b in range(2):
        pltpu.make_async_copy(out[b], num_sp.at[sdstb.at[b]], scs[b]).wait()

    pltpu.sync_copy(den_l, den_sp.at[idx80], add=True)
    plsc.subcore_barrier()

    for j in range(ROWS_PER_TILE // 80):
        r0 = pl.multiple_of(stripe0 + j * 80, 8)
        pltpu.sync_copy(num_sp.at[pl.ds(r0, 80)],
                        num_hbm.at[c, pl.ds(r0, 80)])

    @pl.when(s < DROWS // 8)
    def _():
        r0 = pl.multiple_of(s * 8, 8)
        pltpu.sync_copy(den_sp.at[pl.ds(r0, 8)],
                        den_hbm.at[c, pl.ds(r0, 8)])


def _edge_partials(h, src, dst):
    mesh = plsc.VectorSubcoreMesh(core_axis_name="c", subcore_axis_name="s")
    kfn = pl.kernel(
        _edge_kernel_body,
        out_type=(jax.ShapeDtypeStruct((NC, NPAD, D), jnp.float32),
                  jax.ShapeDtypeStruct((NC, DROWS, D), jnp.float32)),
        mesh=mesh,
        compiler_params=pltpu.CompilerParams(needs_layout_passes=False),
        scratch_types=[
            pltpu.VMEM_SHARED((NPAD, D), jnp.float32),
            pltpu.VMEM_SHARED((DROWS, D), jnp.float32),
            pltpu.VMEM((4, CH), jnp.int32),
            pltpu.VMEM((4, CH), jnp.int32),
            pltpu.VMEM((2, CH), jnp.int32),
            pltpu.VMEM((CH, D), jnp.float32),
            pltpu.VMEM((CH, D), jnp.float32),
            pltpu.VMEM((CH, D), jnp.float32),
            pltpu.VMEM((CH, D), jnp.float32),
            pltpu.VMEM((CH, D), jnp.float32),
            pltpu.VMEM((CH, D), jnp.float32),
            pltpu.VMEM((DROWS, D), jnp.float32),
            pltpu.VMEM((DROWS,), jnp.int32),
            pltpu.VMEM((CH, L), jnp.float32),
        ] + [pltpu.SemaphoreType.DMA] * 14,
    )
    return kfn(h, src, dst)


def _bfdot(a, b):
    return jnp.dot(a.astype(jnp.bfloat16), b.astype(jnp.bfloat16),
                   preferred_element_type=jnp.float32)


def _h_body(x_ref, w_ref, b_ref, o_ref):
    o_ref[...] = _bfdot(x_ref[...], w_ref[...]) + b_ref[...]


def _moe_body(h_ref, wr_ref, w1_ref, b1_ref, w2_ref, b2_ref, o_ref):
    h = h_ref[...]
    logits = _bfdot(h, wr_ref[...])
    m = jnp.max(logits, axis=1, keepdims=True)
    eg = jnp.exp(logits - m)
    g = eg / jnp.sum(eg, axis=1, keepdims=True)
    iota8 = lax.broadcasted_iota(jnp.int32, g.shape, 1)
    m1 = jnp.max(g, axis=1, keepdims=True)
    i1 = jnp.min(jnp.where(g == m1, iota8, 99), axis=1, keepdims=True)
    mask1 = iota8 == i1
    g2 = jnp.where(mask1, -1.0, g)
    m2 = jnp.max(g2, axis=1, keepdims=True)
    i2 = jnp.min(jnp.where(g2 == m2, iota8, 99), axis=1, keepdims=True)
    mask2 = iota8 == i2
    tot = m1 + m2 + 1e-9
    wd = jnp.where(mask1, m1 / tot, 0.0) + jnp.where(mask2, m2 / tot, 0.0)

    w1 = w1_ref[...]
    b1 = b1_ref[...]
    w2 = w2_ref[...]
    b2 = b2_ref[...]
    acc = jnp.zeros(h.shape, jnp.float32)
    for e in range(N_EXP):
        t = jnp.maximum(_bfdot(h, w1[e]) + b1[e:e + 1, :], 0.0)
        o = _bfdot(t, w2[e]) + b2[e:e + 1, :]
        acc = acc + wd[:, e:e + 1] * o
    o_ref[...] = acc


def _fuse_body(num_ref, den_ref, moe_ref, gw1_ref, gb1_ref, gw2_ref, gb2_ref,
               o_ref):
    numpart = num_ref[...]
    denpart = den_ref[...]
    moe = moe_ref[...]
    num = numpart[0] + numpart[1]
    den = denpart[0] + denpart[1]
    gcn = num / (den + 1e-9)
    cat = jnp.concatenate([gcn, moe], axis=1)
    t = jnp.maximum(_bfdot(cat, gw1_ref[...]) + gb1_ref[...], 0.0)
    gl = jnp.sum(t.astype(jnp.bfloat16).astype(jnp.float32)
                 * gw2_ref[...].astype(jnp.bfloat16).astype(jnp.float32),
                 axis=1, keepdims=True) + gb2_ref[0, 0]
    ratio = 1.0 / (1.0 + jnp.exp(-gl))
    gw = 0.1 + 0.9 * ratio
    o_ref[...] = gw * gcn + (1.0 - gw) * moe


def _full(shape):
    return pl.BlockSpec(shape, lambda i: tuple(0 for _ in shape))


BN = 1000


def kernel(x, edge_index, W_in, b_in, Wr, W1, b1, W2, b2, gW1, gb1, gW2, gb2):
    src_i = jnp.asarray(edge_index[0], jnp.int32)
    dst_i = jnp.asarray(edge_index[1], jnp.int32)

    h = pl.pallas_call(
        _h_body,
        grid=(N // BN,),
        in_specs=[pl.BlockSpec((BN, D), lambda i: (i, 0)),
                  _full((D, D)), _full((1, D))],
        out_specs=pl.BlockSpec((BN, D), lambda i: (i, 0)),
        out_shape=jax.ShapeDtypeStruct((N, D), jnp.float32),
    )(x, W_in, b_in.reshape(1, D))

    num_part, den_part = _edge_partials(h, src_i, dst_i)
    den_col = den_part.reshape(NC, NPAD, 1)[:, :N]

    moe = pl.pallas_call(
        _moe_body,
        grid=(N // BN,),
        in_specs=[pl.BlockSpec((BN, D), lambda i: (i, 0)),
                  _full((D, N_EXP)), _full((N_EXP, D, D_FF)),
                  _full((N_EXP, D_FF)), _full((N_EXP, D_FF, D)),
                  _full((N_EXP, D))],
        out_specs=pl.BlockSpec((BN, D), lambda i: (i, 0)),
        out_shape=jax.ShapeDtypeStruct((N, D), jnp.float32),
    )(h, Wr, W1, b1, W2, b2)

    out = pl.pallas_call(
        _fuse_body,
        grid=(N // BN,),
        in_specs=[pl.BlockSpec((NC, BN, D), lambda i: (0, i, 0)),
                  pl.BlockSpec((NC, BN, 1), lambda i: (0, i, 0)),
                  pl.BlockSpec((BN, D), lambda i: (i, 0)),
                  _full((2 * D, D)), _full((1, D)),
                  _full((1, D)), _full((1, 1))],
        out_specs=pl.BlockSpec((BN, D), lambda i: (i, 0)),
        out_shape=jax.ShapeDtypeStruct((N, D), jnp.float32),
    )(num_part, den_col, moe, gW1, gb1.reshape(1, D), gW2.reshape(1, D),
      gb2.reshape(1, 1))

    return out


# speedup vs baseline: 1.5909x; 1.5909x over previous
import functools
import math

import jax
import jax.numpy as jnp
from jax import lax
from jax.experimental import pallas as pl
from jax.experimental.pallas import tpu as pltpu, tpu_sc as plsc

N = 10000
E_EDGES = 320000
D = 128
D_FF = 256
N_EXP = 8
ROWW = D + 16
INV_SQRT_D = 1.0 / math.sqrt(float(D))

NC, NS, L = 2, 16, 16
NW = NC * NS
EDGES_PER_W = E_EDGES // NW
CHUNK = 80
N_CHUNKS = EDGES_PER_W // CHUNK
NPAD = 10240
ROWS_PER_TILE = NPAD // NS
ZROWS = 128


DROWS = NPAD // D
CH = 48
NCH = 212
LAST_OFF = EDGES_PER_W - CH


def _edge_kernel_body(h_hbm, src_hbm, dst_hbm, num_hbm, den_hbm,
                      num_sp, den_sp,
                      srcidx, dstidx, sdstb, hs0, hs1, hd0, hd1,
                      out0, out1, den_l, idx80,
                      sa0, sa1, sb0, sb1, sc0, sc1,
                      sis0, sis1, sis2, sis3, sid0, sid1, sid2, sid3):
    c = lax.axis_index("c")
    s = lax.axis_index("s")
    wid = c * NS + s
    lane = lax.iota(jnp.int32, L)
    hs = [hs0, hs1]
    hd = [hd0, hd1]
    out = [out0, out1]
    sa = [sa0, sa1]
    sb = [sb0, sb1]
    scs = [sc0, sc1]
    sis = [sis0, sis1, sis2, sis3]
    sid = [sid0, sid1, sid2, sid3]

    def zden(r, _):
        for k in range(D // L):
            den_l[r, pl.ds(k * L, L)] = jnp.zeros((L,), jnp.float32)
        return 0
    lax.fori_loop(0, DROWS, zden, 0)

    def zout(r, _):
        for k in range(D // L):
            out0[r, pl.ds(k * L, L)] = jnp.zeros((L,), jnp.float32)
        return 0
    lax.fori_loop(0, CH, zout, 0)

    def fidx(i, _):
        idx80[pl.ds(i * L, L)] = i * L + lane
        return 0
    lax.fori_loop(0, DROWS // L, fidx, 0)

    base = wid * EDGES_PER_W

    def chunk_off(ci):
        return pl.multiple_of(base + jnp.minimum(ci * CH, LAST_OFF), 8)

    for q in range(4):
        pltpu.async_copy(src_hbm.at[pl.ds(chunk_off(q), CH)],
                         srcidx.at[q], sis[q])
        pltpu.async_copy(dst_hbm.at[pl.ds(chunk_off(q), CH)],
                         dstidx.at[q], sid[q])
    for b in range(2):
        pltpu.make_async_copy(src_hbm.at[pl.ds(chunk_off(b), CH)],
                              srcidx.at[b], sis[b]).wait()
        pltpu.make_async_copy(dst_hbm.at[pl.ds(chunk_off(b), CH)],
                              dstidx.at[b], sid[b]).wait()
        pltpu.async_copy(h_hbm.at[srcidx.at[b]], hs[b], sa[b])
        pltpu.async_copy(h_hbm.at[dstidx.at[b]], hd[b], sb[b])

    stripe0 = s * ROWS_PER_TILE
    for j in range(13):
        r0 = pl.multiple_of(stripe0 + j * CH, 8)
        pltpu.sync_copy(out0, num_sp.at[pl.ds(r0, CH)])
    r0 = pl.multiple_of(stripe0 + 13 * CH, 8)
    pltpu.sync_copy(out0.at[pl.ds(0, 16)], num_sp.at[pl.ds(r0, 16)])

    @pl.when(s == 0)
    def _():
        pltpu.sync_copy(out0, den_sp.at[pl.ds(0, CH)])
        pltpu.sync_copy(out0.at[pl.ds(0, 32)], den_sp.at[pl.ds(CH, 32)])

    plsc.subcore_barrier()

    def outer_body(t, _):
        for q in range(4):
            ci = 4 * t + q
            b = q % 2
            pltpu.make_async_copy(h_hbm.at[srcidx.at[q]], hs[b], sa[b]).wait()
            pltpu.make_async_copy(h_hbm.at[dstidx.at[q]], hd[b], sb[b]).wait()

            @pl.when(ci >= 2)
            def _():
                pltpu.make_async_copy(
                    out[b], num_sp.at[sdstb.at[b]], scs[b]).wait()

            for g in range(CH // L):
                sdstb[b, pl.ds(g * L, L)] = dstidx[q, pl.ds(g * L, L)]

            def group_body(g, _, b=b, ci=ci):
                e0 = g * L
                dv = sdstb[b, pl.ds(e0, L)]
                vcond = jnp.logical_or(
                    ci < 208,
                    jnp.logical_and(ci == 208, g == CH // L - 1))
                validf = jnp.where(vcond, 1.0, 0.0)
                for u in range(L):
                    e = e0 + u
                    hsv = [hs[b][e, pl.ds(k * L, L)] for k in range(D // L)]
                    hdv = [hd[b][e, pl.ds(k * L, L)] for k in range(D // L)]
                    acc = hsv[0] * hdv[0]
                    for k in range(1, D // L):
                        acc = acc + hsv[k] * hdv[k]
                    for sh in (8, 4, 2, 1):
                        acc = acc + acc.at[jnp.bitwise_xor(lane, sh)].get(
                            mode="promise_in_bounds", unique_indices=True)
                    wv = jnp.exp(acc * INV_SQRT_D) * validf
                    for k in range(D // L):
                        out[b][e, pl.ds(k * L, L)] = hsv[k] * wv
                    d_e = dv[u]
                    row = lax.shift_right_logical(d_e, 7)
                    seg = lax.shift_left(lax.shift_right_logical(
                        jnp.bitwise_and(d_e, 127), 4), 4)
                    sub = jnp.bitwise_and(d_e, 15)
                    cur = den_l[row, pl.ds(seg, L)]
                    den_l[row, pl.ds(seg, L)] = (
                        cur + jnp.where(lane == sub, wv, 0.0))
                return 0

            lax.fori_loop(0, CH // L, group_body, 0)

            pltpu.async_copy(out[b], num_sp.at[sdstb.at[b]], scs[b],
                             add=True)

            @pl.when(ci + 4 < NCH)
            def _():
                off4 = chunk_off(ci + 4)
                pltpu.async_copy(src_hbm.at[pl.ds(off4, CH)],
                                 srcidx.at[q], sis[q])
                pltpu.async_copy(dst_hbm.at[pl.ds(off4, CH)],
                                 dstidx.at[q], sid[q])

            @pl.when(ci + 2 < NCH)
            def _():
                q2 = (q + 2) % 4
                off2 = chunk_off(ci + 2)
                pltpu.make_async_copy(src_hbm.at[pl.ds(off2, CH)],
                                      srcidx.at[q2], sis[q2]).wait()
                pltpu.make_async_copy(dst_hbm.at[pl.ds(off2, CH)],
                                      dstidx.at[q2], sid[q2]).wait()
                pltpu.async_copy(h_hbm.at[srcidx.at[q2]], hs[b], sa[b])
                pltpu.async_copy(h_hbm.at[dstidx.at[q2]], hd[b], sb[b])
        return 0

    lax.fori_loop(0, NCH // 4, outer_body, 0)
    for b in range(2):
        pltpu.make_async_copy(out[b], num_sp.at[sdstb.at[b]], scs[b]).wait()

    pltpu.sync_copy(den_l, den_sp.at[idx80], add=True)
    plsc.subcore_barrier()

    for j in range(ROWS_PER_TILE // 80):
        r0 = pl.multiple_of(stripe0 + j * 80, 8)
        pltpu.sync_copy(num_sp.at[pl.ds(r0, 80)],
                        num_hbm.at[c, pl.ds(r0, 80)])

    @pl.when(s < DROWS // 8)
    def _():
        r0 = pl.multiple_of(s * 8, 8)
        pltpu.sync_copy(den_sp.at[pl.ds(r0, 8)],
                        den_hbm.at[c, pl.ds(r0, 8)])


def _edge_partials(h, src, dst):
    mesh = plsc.VectorSubcoreMesh(core_axis_name="c", subcore_axis_name="s")
    kfn = pl.kernel(
        _edge_kernel_body,
        out_type=(jax.ShapeDtypeStruct((NC, NPAD, D), jnp.float32),
                  jax.ShapeDtypeStruct((NC, DROWS, D), jnp.float32)),
        mesh=mesh,
        compiler_params=pltpu.CompilerParams(needs_layout_passes=False),
        scratch_types=[
            pltpu.VMEM_SHARED((NPAD, D), jnp.float32),
            pltpu.VMEM_SHARED((DROWS, D), jnp.float32),
            pltpu.VMEM((4, CH), jnp.int32),
            pltpu.VMEM((4, CH), jnp.int32),
            pltpu.VMEM((2, CH), jnp.int32),
            pltpu.VMEM((CH, D), jnp.float32),
            pltpu.VMEM((CH, D), jnp.float32),
            pltpu.VMEM((CH, D), jnp.float32),
            pltpu.VMEM((CH, D), jnp.float32),
            pltpu.VMEM((CH, D), jnp.float32),
            pltpu.VMEM((CH, D), jnp.float32),
            pltpu.VMEM((DROWS, D), jnp.float32),
            pltpu.VMEM((DROWS,), jnp.int32),
        ] + [pltpu.SemaphoreType.DMA] * 14,
    )
    return kfn(h, src, dst)


def _bfdot(a, b):
    return jnp.dot(a.astype(jnp.bfloat16), b.astype(jnp.bfloat16),
                   preferred_element_type=jnp.float32)


def _h_body(x_ref, w_ref, b_ref, o_ref):
    o_ref[...] = _bfdot(x_ref[...], w_ref[...]) + b_ref[...]


def _moe_body(h_ref, wr_ref, w1_ref, b1_ref, w2_ref, b2_ref, o_ref):
    h = h_ref[...]
    logits = _bfdot(h, wr_ref[...])
    m = jnp.max(logits, axis=1, keepdims=True)
    eg = jnp.exp(logits - m)
    g = eg / jnp.sum(eg, axis=1, keepdims=True)
    iota8 = lax.broadcasted_iota(jnp.int32, g.shape, 1)
    m1 = jnp.max(g, axis=1, keepdims=True)
    i1 = jnp.min(jnp.where(g == m1, iota8, 99), axis=1, keepdims=True)
    mask1 = iota8 == i1
    g2 = jnp.where(mask1, -1.0, g)
    m2 = jnp.max(g2, axis=1, keepdims=True)
    i2 = jnp.min(jnp.where(g2 == m2, iota8, 99), axis=1, keepdims=True)
    mask2 = iota8 == i2
    tot = m1 + m2 + 1e-9
    wd = jnp.where(mask1, m1 / tot, 0.0) + jnp.where(mask2, m2 / tot, 0.0)

    w1 = w1_ref[...]
    b1 = b1_ref[...]
    w2 = w2_ref[...]
    b2 = b2_ref[...]
    acc = jnp.zeros(h.shape, jnp.float32)
    for e in range(N_EXP):
        t = jnp.maximum(_bfdot(h, w1[e]) + b1[e:e + 1, :], 0.0)
        o = _bfdot(t, w2[e]) + b2[e:e + 1, :]
        acc = acc + wd[:, e:e + 1] * o
    o_ref[...] = acc


def _fuse_body(num_ref, den_ref, moe_ref, gw1_ref, gb1_ref, gw2_ref, gb2_ref,
               o_ref):
    numpart = num_ref[...]
    denpart = den_ref[...]
    moe = moe_ref[...]
    num = numpart[0] + numpart[1]
    den = denpart[0] + denpart[1]
    gcn = num / (den + 1e-9)
    cat = jnp.concatenate([gcn, moe], axis=1)
    t = jnp.maximum(_bfdot(cat, gw1_ref[...]) + gb1_ref[...], 0.0)
    gl = jnp.sum(t.astype(jnp.bfloat16).astype(jnp.float32)
                 * gw2_ref[...].astype(jnp.bfloat16).astype(jnp.float32),
                 axis=1, keepdims=True) + gb2_ref[0, 0]
    ratio = 1.0 / (1.0 + jnp.exp(-gl))
    gw = 0.1 + 0.9 * ratio
    o_ref[...] = gw * gcn + (1.0 - gw) * moe


def _full(shape):
    return pl.BlockSpec(shape, lambda i: tuple(0 for _ in shape))


BN = 1000


def kernel(x, edge_index, W_in, b_in, Wr, W1, b1, W2, b2, gW1, gb1, gW2, gb2):
    src_i = jnp.asarray(edge_index[0], jnp.int32)
    dst_i = jnp.asarray(edge_index[1], jnp.int32)

    h = pl.pallas_call(
        _h_body,
        grid=(N // BN,),
        in_specs=[pl.BlockSpec((BN, D), lambda i: (i, 0)),
                  _full((D, D)), _full((1, D))],
        out_specs=pl.BlockSpec((BN, D), lambda i: (i, 0)),
        out_shape=jax.ShapeDtypeStruct((N, D), jnp.float32),
    )(x, W_in, b_in.reshape(1, D))

    num_part, den_part = _edge_partials(h, src_i, dst_i)
    den_col = den_part.reshape(NC, NPAD, 1)[:, :N]

    moe = pl.pallas_call(
        _moe_body,
        grid=(N // BN,),
        in_specs=[pl.BlockSpec((BN, D), lambda i: (i, 0)),
                  _full((D, N_EXP)), _full((N_EXP, D, D_FF)),
                  _full((N_EXP, D_FF)), _full((N_EXP, D_FF, D)),
                  _full((N_EXP, D))],
        out_specs=pl.BlockSpec((BN, D), lambda i: (i, 0)),
        out_shape=jax.ShapeDtypeStruct((N, D), jnp.float32),
    )(h, Wr, W1, b1, W2, b2)

    out = pl.pallas_call(
        _fuse_body,
        grid=(N // BN,),
        in_specs=[pl.BlockSpec((NC, BN, D), lambda i: (0, i, 0)),
                  pl.BlockSpec((NC, BN, 1), lambda i: (0, i, 0)),
                  pl.BlockSpec((BN, D), lambda i: (i, 0)),
                  _full((2 * D, D)), _full((1, D)),
                  _full((1, D)), _full((1, 1))],
        out_specs=pl.BlockSpec((BN, D), lambda i: (i, 0)),
        out_shape=jax.ShapeDtypeStruct((N, D), jnp.float32),
    )(num_part, den_col, moe, gW1, gb1.reshape(1, D), gW2.reshape(1, D),
      gb2.reshape(1, 1))

    return out
